# baseline (speedup 1.0000x reference)
import functools

import jax
import jax.numpy as jnp
from jax import lax
from jax.experimental import pallas as pl
from jax.experimental.pallas import tpu as pltpu
from jax.experimental.pallas import tpu_sc as plsc

NC = 2
NS = 16
NW = NC * NS
LANE = 128
CR = 130


def _stats_body(x_ref, o_ref):
    a = x_ref[...]
    col = lax.broadcasted_iota(jnp.int32, a.shape, 1)
    even = (col % 2 == 0).astype(jnp.float32)
    ae = a * even
    aa = a * a
    s0 = jnp.sum(ae)
    s00 = jnp.sum(aa * even)
    b = pltpu.roll(a, a.shape[1] - 1, 1)
    o_ref[0] = s0
    o_ref[1] = jnp.sum(a) - s0
    o_ref[2] = s00
    o_ref[3] = jnp.sum(a * b * even)
    o_ref[4] = jnp.sum(aa) - s00


def _main_body(n_rows, stats_ref, pa_ref, x_ref, face_ref, we1_ref, be1_ref,
               gam_ref, bet_ref, we2_ref, be2_ref, w0_ref, wg_ref, wface_ref,
               wf_ref, s_out, lin_out):
    ninv = 1.0 / float(n_rows)
    s0 = stats_ref[0] * ninv
    s1 = stats_ref[1] * ninv
    c00 = stats_ref[2] * ninv - s0 * s0
    c01 = stats_ref[3] * ninv - s0 * s1
    c11 = stats_ref[4] * ninv - s1 * s1
    w0r = we1_ref[0:1, :]
    w1r = we1_ref[1:2, :]
    mu = s0 * w0r + s1 * w1r + be1_ref[...]
    var = w0r * w0r * c00 + 2.0 * w0r * w1r * c01 + w1r * w1r * c11
    inv = lax.rsqrt(var + 1e-5)
    h = jnp.dot(x_ref[...], we1_ref[...],
                preferred_element_type=jnp.float32) + be1_ref[...]
    hn = (h - mu) * (inv * gam_ref[...]) + bet_ref[...]
    a = pa_ref[0]
    hp = jnp.where(hn >= 0, hn, a * hn)
    e = jnp.dot(hp, we2_ref[...],
                preferred_element_type=jnp.float32) + be2_ref[...]
    lin = jnp.dot(e, w0_ref[...], preferred_element_type=jnp.float32)
    lin_out[...] = lin.reshape(lin_out.shape)
    m = jnp.dot(e, wg_ref[...], preferred_element_type=jnp.float32)
    m = m + jnp.dot(face_ref[...], wface_ref[...],
                    preferred_element_type=jnp.float32)
    sv = jnp.dot(m, wf_ref[...], preferred_element_type=jnp.float32)
    s_out[...] = sv.reshape(s_out.shape)


def _post_body(hist_ref, s_ref, dinv_ref, t_ref):
    deg = hist_ref[0] + hist_ref[1] + 1.0
    dinv = lax.rsqrt(deg)
    dinv_ref[...] = dinv
    t_ref[...] = s_ref[...] * dinv


def _final_body(lin_ref, s_ref, dinv_ref, acc_ref, bg_ref, wft_ref, bb_ref,
                out_ref):
    const = jnp.sum(bg_ref[...] * wft_ref[...]) + bb_ref[0] + bb_ref[1]
    dinv = dinv_ref[...]
    out_ref[...] = (lin_ref[...] + (acc_ref[0] + acc_ref[1]) * dinv
                    + s_ref[...] * dinv * dinv + const)


def _make_hist_sc(np_pad, rows_main, n_chunks, tail_rows):
    mesh = plsc.VectorSubcoreMesh(core_axis_name="c", subcore_axis_name="s")

    @functools.partial(
        pl.kernel,
        out_type=jax.ShapeDtypeStruct((NC, np_pad), jnp.float32),
        mesh=mesh,
        scratch_types=[
            pltpu.VMEM((CR * LANE,), jnp.int32),
            pltpu.VMEM((CR * LANE,), jnp.float32),
            pltpu.VMEM((LANE,), jnp.int32),
            pltpu.VMEM((LANE,), jnp.float32),
            pltpu.VMEM_SHARED((np_pad,), jnp.float32),
        ],
    )
    def hist_sc(dst_hbm, ones_hbm, zeros_np, out, idx_v, ones_v, tidx_v,
                tones_v, hist_s):
        c = lax.axis_index("c")
        s = lax.axis_index("s")
        wid = c * NS + s

        @pl.when(s == 0)
        def _():
            pltpu.sync_copy(zeros_np, hist_s)

        pltpu.sync_copy(ones_hbm, ones_v)
        pltpu.sync_copy(ones_hbm.at[pl.ds(0, LANE)], tones_v)
        plsc.subcore_barrier()

        base = wid * rows_main * LANE

        def chunk(i, carry):
            off = base + i * (CR * LANE)
            pltpu.sync_copy(dst_hbm.at[pl.ds(off, CR * LANE)], idx_v)
            pltpu.sync_copy(ones_v, hist_s.at[idx_v], add=True)
            return carry

        lax.fori_loop(0, n_chunks, chunk, 0)

        @pl.when(wid < tail_rows)
        def _():
            toff = (NW * rows_main + wid) * LANE
            pltpu.sync_copy(dst_hbm.at[pl.ds(toff, LANE)], tidx_v)
            pltpu.sync_copy(tones_v, hist_s.at[tidx_v], add=True)

        plsc.subcore_barrier()

        @pl.when(s == 0)
        def _():
            pltpu.sync_copy(hist_s, out.at[c])

    return hist_sc


def _make_edge_sc(np_pad, rows_main, n_chunks, tail_rows):
    mesh = plsc.VectorSubcoreMesh(core_axis_name="c", subcore_axis_name="s")

    @functools.partial(
        pl.kernel,
        out_type=jax.ShapeDtypeStruct((NC, np_pad), jnp.float32),
        mesh=mesh,
        scratch_types=[
            pltpu.VMEM((CR * LANE,), jnp.int32),
            pltpu.VMEM((CR * LANE,), jnp.int32),
            pltpu.VMEM((CR * LANE,), jnp.float32),
            pltpu.VMEM((LANE,), jnp.int32),
            pltpu.VMEM((LANE,), jnp.int32),
            pltpu.VMEM((LANE,), jnp.float32),
            pltpu.VMEM_SHARED((np_pad,), jnp.float32),
            pltpu.VMEM_SHARED((np_pad,), jnp.float32),
        ],
    )
    def edge_sc(src_hbm, dst_hbm, t_hbm, zeros_np, out, sidx_v, didx_v,
                vals_v, tsidx_v, tdidx_v, tvals_v, t_s, acc_s):
        c = lax.axis_index("c")
        s = lax.axis_index("s")
        wid = c * NS + s

        @pl.when(s == 0)
        def _():
            pltpu.sync_copy(zeros_np, acc_s)

        @pl.when(s == 1)
        def _():
            pltpu.sync_copy(t_hbm, t_s)

        plsc.subcore_barrier()
        base = wid * rows_main * LANE

        def chunk(i, carry):
            off = base + i * (CR * LANE)
            pltpu.sync_copy(src_hbm.at[pl.ds(off, CR * LANE)], sidx_v)
            pltpu.sync_copy(dst_hbm.at[pl.ds(off, CR * LANE)], didx_v)
            pltpu.sync_copy(t_s.at[sidx_v], vals_v)
            pltpu.sync_copy(vals_v, acc_s.at[didx_v], add=True)
            return carry

        lax.fori_loop(0, n_chunks, chunk, 0)

        @pl.when(wid < tail_rows)
        def _():
            toff = (NW * rows_main + wid) * LANE
            pltpu.sync_copy(src_hbm.at[pl.ds(toff, LANE)], tsidx_v)
            pltpu.sync_copy(dst_hbm.at[pl.ds(toff, LANE)], tdidx_v)
            pltpu.sync_copy(t_s.at[tsidx_v], tvals_v)
            pltpu.sync_copy(tvals_v, acc_s.at[tdidx_v], add=True)

        plsc.subcore_barrier()

        @pl.when(s == 0)
        def _():
            pltpu.sync_copy(acc_s, out.at[c])

    return edge_sc


def kernel(x, edge_index, face_feats, W_e1, b_e1, bn_gamma, bn_beta, prelu_a,
           W_e2, b_e2, W0, b0, Wg, bg, Wface, Wf, bf):
    n = x.shape[0]
    e = edge_index.shape[1]
    ff = face_feats.shape[1]
    np_pad = ((n + 8 * LANE - 1) // (8 * LANE)) * (8 * LANE)
    rows2 = np_pad // LANE

    e_rows = e // LANE
    rows_main = (e_rows // NW // CR) * CR
    n_chunks = rows_main // CR
    tail_rows = e_rows - NW * rows_main
    assert rows_main * NW + tail_rows == e_rows and tail_rows <= NW

    bn = 7168
    brows = bn // LANE
    n_blocks = np_pad // bn

    f32 = jnp.float32
    b_e1r = b_e1.reshape(1, -1)
    gam = bn_gamma.reshape(1, -1)
    bet = bn_beta.reshape(1, -1)
    b_e2r = b_e2.reshape(1, -1)
    bgr = bg.reshape(1, -1)
    wft = Wf.reshape(1, -1)
    bb = jnp.concatenate([bf, b0]).astype(f32)
    zeros_np = jnp.zeros((np_pad,), f32)
    ones_cr = jnp.ones((CR * LANE,), f32)
    src1 = edge_index[0]
    dst1 = edge_index[1]
    xflat = x.reshape(-1)
    xpad_len = ((2 * n + LANE - 1) // LANE) * LANE
    x2 = jnp.pad(xflat, (0, xpad_len - 2 * n)).reshape(-1, LANE)

    stats = pl.pallas_call(
        _stats_body,
        out_specs=pl.BlockSpec(memory_space=pltpu.SMEM),
        out_shape=jax.ShapeDtypeStruct((8,), f32),
    )(x2)

    hist = _make_hist_sc(np_pad, rows_main, n_chunks, tail_rows)(
        dst1, ones_cr, zeros_np)

    wspec = lambda shp: pl.BlockSpec(shp, lambda i: (0, 0))
    s_col, lin_col = pl.pallas_call(
        functools.partial(_main_body, n),
        grid=(n_blocks,),
        in_specs=[
            pl.BlockSpec(memory_space=pltpu.SMEM),
            pl.BlockSpec(memory_space=pltpu.SMEM),
            pl.BlockSpec((bn, 2), lambda i: (i, 0)),
            pl.BlockSpec((bn, ff), lambda i: (i, 0)),
            wspec((2, 32)), wspec((1, 32)), wspec((1, 32)), wspec((1, 32)),
            wspec((32, 32)), wspec((1, 32)), wspec((32, 1)), wspec((32, 32)),
            wspec((ff, 32)), wspec((32, 1)),
        ],
        out_specs=[
            pl.BlockSpec((brows, LANE), lambda i: (i, 0)),
            pl.BlockSpec((brows, LANE), lambda i: (i, 0)),
        ],
        out_shape=[
            jax.ShapeDtypeStruct((rows2, LANE), f32),
            jax.ShapeDtypeStruct((rows2, LANE), f32),
        ],
    )(stats, prelu_a, x, face_feats, W_e1, b_e1r, gam, bet, W_e2, b_e2r,
      W0, Wg, Wface, Wf)

    s2, lin2 = s_col, lin_col
    dinv2, t2 = pl.pallas_call(
        _post_body,
        out_shape=[
            jax.ShapeDtypeStruct((rows2, LANE), f32),
            jax.ShapeDtypeStruct((rows2, LANE), f32),
        ],
    )(hist.reshape(NC, rows2, LANE), s2)

    acc = _make_edge_sc(np_pad, rows_main, n_chunks, tail_rows)(
        src1, dst1, t2.reshape(np_pad), zeros_np)

    out2 = pl.pallas_call(
        _final_body,
        out_shape=jax.ShapeDtypeStruct((rows2, LANE), f32),
        in_specs=[
            pl.BlockSpec((rows2, LANE), lambda: (0, 0)),
            pl.BlockSpec((rows2, LANE), lambda: (0, 0)),
            pl.BlockSpec((rows2, LANE), lambda: (0, 0)),
            pl.BlockSpec((NC, rows2, LANE), lambda: (0, 0, 0)),
            pl.BlockSpec((1, 32), lambda: (0, 0)),
            pl.BlockSpec((1, 32), lambda: (0, 0)),
            pl.BlockSpec(memory_space=pltpu.SMEM),
        ],
    )(lin2, s2, dinv2, acc.reshape(NC, rows2, LANE), bgr, wft, bb)

    return out2.reshape(np_pad)[:n]


# speedup vs baseline: 1.3131x; 1.3131x over previous
import functools

import jax
import jax.numpy as jnp
from jax import lax
from jax.experimental import pallas as pl
from jax.experimental.pallas import tpu as pltpu
from jax.experimental.pallas import tpu_sc as plsc

NC = 2
NS = 16
NW = NC * NS
LANE = 128
CR = 130


def _main_body(n_rows, bn, pa_ref, x_ref, face_ref, we1_ref, be1_ref,
               gam_ref, bet_ref, we2_ref, be2_ref, w0_ref, wg_ref, wface_ref,
               wf_ref, s_out, lin_out, acc_ref):
    ph = pl.program_id(0)
    i = pl.program_id(1)

    @pl.when((ph == 0) & (i == 0))
    def _():
        for k in range(5):
            acc_ref[k] = 0.0

    @pl.when(ph == 0)
    def _():
        xb = x_ref[...]
        row = lax.broadcasted_iota(jnp.int32, xb.shape, 0) + i * bn
        xb = jnp.where(row < n_rows, xb, 0.0)
        x0 = xb[:, 0:1]
        x1 = xb[:, 1:2]
        acc_ref[0] += jnp.sum(x0)
        acc_ref[1] += jnp.sum(x1)
        acc_ref[2] += jnp.sum(x0 * x0)
        acc_ref[3] += jnp.sum(x0 * x1)
        acc_ref[4] += jnp.sum(x1 * x1)

    @pl.when(ph == 1)
    def _():
        ninv = 1.0 / float(n_rows)
        s0 = acc_ref[0] * ninv
        s1 = acc_ref[1] * ninv
        c00 = acc_ref[2] * ninv - s0 * s0
        c01 = acc_ref[3] * ninv - s0 * s1
        c11 = acc_ref[4] * ninv - s1 * s1
        w0r = we1_ref[0:1, :]
        w1r = we1_ref[1:2, :]
        mu = s0 * w0r + s1 * w1r + be1_ref[...]
        var = w0r * w0r * c00 + 2.0 * w0r * w1r * c01 + w1r * w1r * c11
        inv = lax.rsqrt(var + 1e-5)
        h = jnp.dot(x_ref[...], we1_ref[...],
                    preferred_element_type=jnp.float32) + be1_ref[...]
        hn = (h - mu) * (inv * gam_ref[...]) + bet_ref[...]
        a = pa_ref[0]
        hp = jnp.where(hn >= 0, hn, a * hn)
        e = jnp.dot(hp, we2_ref[...],
                    preferred_element_type=jnp.float32) + be2_ref[...]
        lin = jnp.dot(e, w0_ref[...], preferred_element_type=jnp.float32)
        lin_out[...] = lin.reshape(lin_out.shape)
        m = jnp.dot(e, wg_ref[...], preferred_element_type=jnp.float32)
        m = m + jnp.dot(face_ref[...], wface_ref[...],
                        preferred_element_type=jnp.float32)
        sv = jnp.dot(m, wf_ref[...], preferred_element_type=jnp.float32)
        s_out[...] = sv.reshape(s_out.shape)


def _post_body(hist_ref, s_ref, dinv_ref, t_ref):
    deg = hist_ref[0] + hist_ref[1] + 1.0
    dinv = lax.rsqrt(deg)
    dinv_ref[...] = dinv
    t_ref[...] = s_ref[...] * dinv


def _final_body(lin_ref, s_ref, dinv_ref, acc_ref, bg_ref, wft_ref, bb_ref,
                out_ref):
    const = jnp.sum(bg_ref[...] * wft_ref[...]) + bb_ref[0] + bb_ref[1]
    dinv = dinv_ref[...]
    out_ref[...] = (lin_ref[...] + (acc_ref[0] + acc_ref[1]) * dinv
                    + s_ref[...] * dinv * dinv + const)


def _make_hist_sc(np_pad, rows_main, n_chunks, tail_rows):
    mesh = plsc.VectorSubcoreMesh(core_axis_name="c", subcore_axis_name="s")

    @functools.partial(
        pl.kernel,
        out_type=jax.ShapeDtypeStruct((NC, np_pad), jnp.float32),
        mesh=mesh,
        scratch_types=[
            pltpu.VMEM((CR * LANE,), jnp.int32),
            pltpu.VMEM((CR * LANE,), jnp.float32),
            pltpu.VMEM((LANE,), jnp.int32),
            pltpu.VMEM((LANE,), jnp.float32),
            pltpu.VMEM_SHARED((np_pad,), jnp.float32),
        ],
    )
    def hist_sc(ei, ones_hbm, zeros_np, out, idx_v, ones_v, tidx_v,
                tones_v, hist_s):
        c = lax.axis_index("c")
        s = lax.axis_index("s")
        wid = c * NS + s

        @pl.when(s == 0)
        def _():
            pltpu.sync_copy(zeros_np, hist_s)

        pltpu.sync_copy(ones_hbm, ones_v)
        pltpu.sync_copy(ones_hbm.at[pl.ds(0, LANE)], tones_v)
        plsc.subcore_barrier()

        base = wid * rows_main * LANE

        def chunk(i, carry):
            off = base + i * (CR * LANE)
            pltpu.sync_copy(ei.at[1, pl.ds(off, CR * LANE)], idx_v)
            pltpu.sync_copy(ones_v, hist_s.at[idx_v], add=True)
            return carry

        lax.fori_loop(0, n_chunks, chunk, 0)

        @pl.when(wid < tail_rows)
        def _():
            toff = (NW * rows_main + wid) * LANE
            pltpu.sync_copy(ei.at[1, pl.ds(toff, LANE)], tidx_v)
            pltpu.sync_copy(tones_v, hist_s.at[tidx_v], add=True)

        plsc.subcore_barrier()

        @pl.when(s == 0)
        def _():
            pltpu.sync_copy(hist_s, out.at[c])

    return hist_sc


def _make_edge_sc(np_pad, rows_main, n_chunks, tail_rows):
    mesh = plsc.VectorSubcoreMesh(core_axis_name="c", subcore_axis_name="s")

    @functools.partial(
        pl.kernel,
        out_type=jax.ShapeDtypeStruct((NC, np_pad), jnp.float32),
        mesh=mesh,
        scratch_types=[
            pltpu.VMEM((CR * LANE,), jnp.int32),
            pltpu.VMEM((CR * LANE,), jnp.int32),
            pltpu.VMEM((CR * LANE,), jnp.float32),
            pltpu.VMEM((LANE,), jnp.int32),
            pltpu.VMEM((LANE,), jnp.int32),
            pltpu.VMEM((LANE,), jnp.float32),
            pltpu.VMEM_SHARED((np_pad,), jnp.float32),
            pltpu.VMEM_SHARED((np_pad,), jnp.float32),
        ],
    )
    def edge_sc(ei, t_hbm, zeros_np, out, sidx_v, didx_v,
                vals_v, tsidx_v, tdidx_v, tvals_v, t_s, acc_s):
        c = lax.axis_index("c")
        s = lax.axis_index("s")
        wid = c * NS + s

        @pl.when(s == 0)
        def _():
            pltpu.sync_copy(zeros_np, acc_s)

        @pl.when(s == 1)
        def _():
            pltpu.sync_copy(t_hbm, t_s)

        plsc.subcore_barrier()
        base = wid * rows_main * LANE

        def chunk(i, carry):
            off = base + i * (CR * LANE)
            pltpu.sync_copy(ei.at[0, pl.ds(off, CR * LANE)], sidx_v)
            pltpu.sync_copy(ei.at[1, pl.ds(off, CR * LANE)], didx_v)
            pltpu.sync_copy(t_s.at[sidx_v], vals_v)
            pltpu.sync_copy(vals_v, acc_s.at[didx_v], add=True)
            return carry

        lax.fori_loop(0, n_chunks, chunk, 0)

        @pl.when(wid < tail_rows)
        def _():
            toff = (NW * rows_main + wid) * LANE
            pltpu.sync_copy(ei.at[0, pl.ds(toff, LANE)], tsidx_v)
            pltpu.sync_copy(ei.at[1, pl.ds(toff, LANE)], tdidx_v)
            pltpu.sync_copy(t_s.at[tsidx_v], tvals_v)
            pltpu.sync_copy(tvals_v, acc_s.at[tdidx_v], add=True)

        plsc.subcore_barrier()

        @pl.when(s == 0)
        def _():
            pltpu.sync_copy(acc_s, out.at[c])

    return edge_sc


def kernel(x, edge_index, face_feats, W_e1, b_e1, bn_gamma, bn_beta, prelu_a,
           W_e2, b_e2, W0, b0, Wg, bg, Wface, Wf, bf):
    n = x.shape[0]
    e = edge_index.shape[1]
    ff = face_feats.shape[1]
    np_pad = ((n + 8 * LANE - 1) // (8 * LANE)) * (8 * LANE)
    rows2 = np_pad // LANE

    e_rows = e // LANE
    rows_main = (e_rows // NW // CR) * CR
    n_chunks = rows_main // CR
    tail_rows = e_rows - NW * rows_main
    assert rows_main * NW + tail_rows == e_rows and tail_rows <= NW

    bn = 7168
    brows = bn // LANE
    n_blocks = np_pad // bn

    f32 = jnp.float32
    b_e1r = b_e1.reshape(1, -1)
    gam = bn_gamma.reshape(1, -1)
    bet = bn_beta.reshape(1, -1)
    b_e2r = b_e2.reshape(1, -1)
    bgr = bg.reshape(1, -1)
    wft = Wf.reshape(1, -1)
    bb = jnp.concatenate([bf, b0]).astype(f32)
    zeros_np = jnp.zeros((np_pad,), f32)
    ones_cr = jnp.ones((CR * LANE,), f32)

    hist = _make_hist_sc(np_pad, rows_main, n_chunks, tail_rows)(
        edge_index, ones_cr, zeros_np)

    wspec = lambda shp: pl.BlockSpec(shp, lambda p, i: (0, 0))
    s_col, lin_col = pl.pallas_call(
        functools.partial(_main_body, n, bn),
        grid=(2, n_blocks),
        in_specs=[
            pl.BlockSpec(memory_space=pltpu.SMEM),
            pl.BlockSpec((bn, 2), lambda p, i: (i, 0)),
            pl.BlockSpec((bn, ff), lambda p, i: (i * p, 0)),
            wspec((2, 32)), wspec((1, 32)), wspec((1, 32)), wspec((1, 32)),
            wspec((32, 32)), wspec((1, 32)), wspec((32, 1)), wspec((32, 32)),
            wspec((ff, 32)), wspec((32, 1)),
        ],
        out_specs=[
            pl.BlockSpec((brows, LANE), lambda p, i: (i, 0)),
            pl.BlockSpec((brows, LANE), lambda p, i: (i, 0)),
        ],
        out_shape=[
            jax.ShapeDtypeStruct((rows2, LANE), f32),
            jax.ShapeDtypeStruct((rows2, LANE), f32),
        ],
        scratch_shapes=[pltpu.SMEM((8,), f32)],
    )(prelu_a, x, face_feats, W_e1, b_e1r, gam, bet, W_e2, b_e2r,
      W0, Wg, Wface, Wf)

    s2, lin2 = s_col, lin_col
    dinv2, t2 = pl.pallas_call(
        _post_body,
        out_shape=[
            jax.ShapeDtypeStruct((rows2, LANE), f32),
            jax.ShapeDtypeStruct((rows2, LANE), f32),
        ],
    )(hist.reshape(NC, rows2, LANE), s2)

    acc = _make_edge_sc(np_pad, rows_main, n_chunks, tail_rows)(
        edge_index, t2.reshape(np_pad), zeros_np)

    out2 = pl.pallas_call(
        _final_body,
        out_shape=jax.ShapeDtypeStruct((rows2, LANE), f32),
        in_specs=[
            pl.BlockSpec((rows2, LANE), lambda: (0, 0)),
            pl.BlockSpec((rows2, LANE), lambda: (0, 0)),
            pl.BlockSpec((rows2, LANE), lambda: (0, 0)),
            pl.BlockSpec((NC, rows2, LANE), lambda: (0, 0, 0)),
            pl.BlockSpec((1, 32), lambda: (0, 0)),
            pl.BlockSpec((1, 32), lambda: (0, 0)),
            pl.BlockSpec(memory_space=pltpu.SMEM),
        ],
    )(lin2, s2, dinv2, acc.reshape(NC, rows2, LANE), bgr, wft, bb)

    return out2.reshape(np_pad)[:n]


# speedup vs baseline: 1.3889x; 1.0578x over previous
import functools

import jax
import jax.numpy as jnp
from jax import lax
from jax.experimental import pallas as pl
from jax.experimental.pallas import tpu as pltpu
from jax.experimental.pallas import tpu_sc as plsc

NC = 2
NS = 16
NW = NC * NS
LANE = 128
CR = 130


def _main_body(n_rows, bn, pa_ref, x_ref, face_ref, we1_ref, be1_ref,
               gam_ref, bet_ref, we2_ref, be2_ref, w0_ref, wg_ref, wface_ref,
               wf_ref, s_out, lin_out, acc_ref):
    ph = pl.program_id(0)
    i = pl.program_id(1)

    @pl.when((ph == 0) & (i == 0))
    def _():
        for k in range(5):
            acc_ref[k] = 0.0

    @pl.when(ph == 0)
    def _():
        xb = x_ref[...]
        row = lax.broadcasted_iota(jnp.int32, xb.shape, 0) + i * bn
        valid = row < n_rows
        xa = jnp.concatenate(
            [jnp.where(valid, xb, 0.0),
             jnp.where(valid[:, 0:1], 1.0, 0.0)], axis=1)
        g = lax.dot_general(xa, xa, (((0,), (0,)), ((), ())),
                            preferred_element_type=jnp.float32)
        acc_ref[0] += g[0, 2]
        acc_ref[1] += g[1, 2]
        acc_ref[2] += g[0, 0]
        acc_ref[3] += g[0, 1]
        acc_ref[4] += g[1, 1]

    @pl.when(ph == 1)
    def _():
        ninv = 1.0 / float(n_rows)
        s0 = acc_ref[0] * ninv
        s1 = acc_ref[1] * ninv
        c00 = acc_ref[2] * ninv - s0 * s0
        c01 = acc_ref[3] * ninv - s0 * s1
        c11 = acc_ref[4] * ninv - s1 * s1
        w0r = we1_ref[0:1, :]
        w1r = we1_ref[1:2, :]
        mu = s0 * w0r + s1 * w1r + be1_ref[...]
        var = w0r * w0r * c00 + 2.0 * w0r * w1r * c01 + w1r * w1r * c11
        inv = lax.rsqrt(var + 1e-5)
        h = jnp.dot(x_ref[...], we1_ref[...],
                    preferred_element_type=jnp.float32) + be1_ref[...]
        hn = (h - mu) * (inv * gam_ref[...]) + bet_ref[...]
        a = pa_ref[0]
        hp = jnp.where(hn >= 0, hn, a * hn)
        e = jnp.dot(hp, we2_ref[...],
                    preferred_element_type=jnp.float32) + be2_ref[...]
        lin = jnp.dot(e, w0_ref[...], preferred_element_type=jnp.float32)
        lin_out[...] = lin.reshape(lin_out.shape)
        m = jnp.dot(e, wg_ref[...], preferred_element_type=jnp.float32)
        m = m + jnp.dot(face_ref[...], wface_ref[...],
                        preferred_element_type=jnp.float32)
        sv = jnp.dot(m, wf_ref[...], preferred_element_type=jnp.float32)
        s_out[...] = sv.reshape(s_out.shape)


def _post_body(hist_ref, s_ref, dinv_ref, t_ref):
    deg = hist_ref[0] + hist_ref[1] + 1.0
    dinv = lax.rsqrt(deg)
    dinv_ref[...] = dinv
    t_ref[...] = s_ref[...] * dinv


def _final_body(lin_ref, s_ref, dinv_ref, acc_ref, bg_ref, wft_ref, bb_ref,
                out_ref):
    const = jnp.sum(bg_ref[...] * wft_ref[...]) + bb_ref[0] + bb_ref[1]
    dinv = dinv_ref[...]
    out_ref[...] = (lin_ref[...] + (acc_ref[0] + acc_ref[1]) * dinv
                    + s_ref[...] * dinv * dinv + const)


def _make_hist_sc(np_pad, rows_main, n_chunks, tail_rows):
    mesh = plsc.VectorSubcoreMesh(core_axis_name="c", subcore_axis_name="s")

    @functools.partial(
        pl.kernel,
        out_type=jax.ShapeDtypeStruct((NC, np_pad), jnp.float32),
        mesh=mesh,
        scratch_types=[
            pltpu.VMEM((CR * LANE,), jnp.int32),
            pltpu.VMEM((CR * LANE,), jnp.float32),
            pltpu.VMEM((LANE,), jnp.int32),
            pltpu.VMEM((LANE,), jnp.float32),
            pltpu.VMEM_SHARED((np_pad,), jnp.float32),
        ],
    )
    def hist_sc(ei, ones_hbm, zeros_np, out, idx_v, ones_v, tidx_v,
                tones_v, hist_s):
        c = lax.axis_index("c")
        s = lax.axis_index("s")
        wid = c * NS + s

        @pl.when(s == 0)
        def _():
            pltpu.sync_copy(zeros_np, hist_s)

        pltpu.sync_copy(ones_hbm, ones_v)
        pltpu.sync_copy(ones_hbm.at[pl.ds(0, LANE)], tones_v)
        plsc.subcore_barrier()

        base = wid * rows_main * LANE

        def chunk(i, carry):
            off = base + i * (CR * LANE)
            pltpu.sync_copy(ei.at[1, pl.ds(off, CR * LANE)], idx_v)
            pltpu.sync_copy(ones_v, hist_s.at[idx_v], add=True)
            return carry

        lax.fori_loop(0, n_chunks, chunk, 0)

        @pl.when(wid < tail_rows)
        def _():
            toff = (NW * rows_main + wid) * LANE
            pltpu.sync_copy(ei.at[1, pl.ds(toff, LANE)], tidx_v)
            pltpu.sync_copy(tones_v, hist_s.at[tidx_v], add=True)

        plsc.subcore_barrier()

        @pl.when(s == 0)
        def _():
            pltpu.sync_copy(hist_s, out.at[c])

    return hist_sc


def _make_edge_sc(np_pad, rows_main, n_chunks, tail_rows):
    mesh = plsc.VectorSubcoreMesh(core_axis_name="c", subcore_axis_name="s")

    @functools.partial(
        pl.kernel,
        out_type=jax.ShapeDtypeStruct((NC, np_pad), jnp.float32),
        mesh=mesh,
        scratch_types=[
            pltpu.VMEM((CR * LANE,), jnp.int32),
            pltpu.VMEM((CR * LANE,), jnp.int32),
            pltpu.VMEM((CR * LANE,), jnp.float32),
            pltpu.VMEM((LANE,), jnp.int32),
            pltpu.VMEM((LANE,), jnp.int32),
            pltpu.VMEM((LANE,), jnp.float32),
            pltpu.VMEM_SHARED((np_pad,), jnp.float32),
            pltpu.VMEM_SHARED((np_pad,), jnp.float32),
        ],
    )
    def edge_sc(ei, t_hbm, zeros_np, out, sidx_v, didx_v,
                vals_v, tsidx_v, tdidx_v, tvals_v, t_s, acc_s):
        c = lax.axis_index("c")
        s = lax.axis_index("s")
        wid = c * NS + s

        @pl.when(s == 0)
        def _():
            pltpu.sync_copy(zeros_np, acc_s)

        @pl.when(s == 1)
        def _():
            pltpu.sync_copy(t_hbm, t_s)

        plsc.subcore_barrier()
        base = wid * rows_main * LANE

        def chunk(i, carry):
            off = base + i * (CR * LANE)
            pltpu.sync_copy(ei.at[0, pl.ds(off, CR * LANE)], sidx_v)
            pltpu.sync_copy(ei.at[1, pl.ds(off, CR * LANE)], didx_v)
            pltpu.sync_copy(t_s.at[sidx_v], vals_v)
            pltpu.sync_copy(vals_v, acc_s.at[didx_v], add=True)
            return carry

        lax.fori_loop(0, n_chunks, chunk, 0)

        @pl.when(wid < tail_rows)
        def _():
            toff = (NW * rows_main + wid) * LANE
            pltpu.sync_copy(ei.at[0, pl.ds(toff, LANE)], tsidx_v)
            pltpu.sync_copy(ei.at[1, pl.ds(toff, LANE)], tdidx_v)
            pltpu.sync_copy(t_s.at[tsidx_v], tvals_v)
            pltpu.sync_copy(tvals_v, acc_s.at[tdidx_v], add=True)

        plsc.subcore_barrier()

        @pl.when(s == 0)
        def _():
            pltpu.sync_copy(acc_s, out.at[c])

    return edge_sc


def kernel(x, edge_index, face_feats, W_e1, b_e1, bn_gamma, bn_beta, prelu_a,
           W_e2, b_e2, W0, b0, Wg, bg, Wface, Wf, bf):
    n = x.shape[0]
    e = edge_index.shape[1]
    ff = face_feats.shape[1]
    np_pad = ((n + 8 * LANE - 1) // (8 * LANE)) * (8 * LANE)
    rows2 = np_pad // LANE

    e_rows = e // LANE
    rows_main = (e_rows // NW // CR) * CR
    n_chunks = rows_main // CR
    tail_rows = e_rows - NW * rows_main
    assert rows_main * NW + tail_rows == e_rows and tail_rows <= NW

    bn = 7168
    brows = bn // LANE
    n_blocks = np_pad // bn
    hrows = bn * 32 // LANE
    hrows2 = np_pad * 32 // LANE

    f32 = jnp.float32
    b_e1r = b_e1.reshape(1, -1)
    gam = bn_gamma.reshape(1, -1)
    bet = bn_beta.reshape(1, -1)
    b_e2r = b_e2.reshape(1, -1)
    bgr = bg.reshape(1, -1)
    wft = Wf.reshape(1, -1)
    bb = jnp.concatenate([bf, b0]).astype(f32)
    zeros_np = jnp.zeros((np_pad,), f32)
    ones_cr = jnp.ones((CR * LANE,), f32)

    hist = _make_hist_sc(np_pad, rows_main, n_chunks, tail_rows)(
        edge_index, ones_cr, zeros_np)

    wspec = lambda shp: pl.BlockSpec(shp, lambda p, i: (0, 0))
    s_col, lin_col = pl.pallas_call(
        functools.partial(_main_body, n, bn),
        grid=(2, n_blocks),
        in_specs=[
            pl.BlockSpec(memory_space=pltpu.SMEM),
            pl.BlockSpec((bn, 2), lambda p, i: (i, 0)),
            pl.BlockSpec((bn, ff), lambda p, i: (i * p, 0)),
            wspec((2, 32)), wspec((1, 32)), wspec((1, 32)), wspec((1, 32)),
            wspec((32, 32)), wspec((1, 32)), wspec((32, 1)), wspec((32, 32)),
            wspec((ff, 32)), wspec((32, 1)),
        ],
        out_specs=[
            pl.BlockSpec((brows, LANE), lambda p, i: (i, 0)),
            pl.BlockSpec((brows, LANE), lambda p, i: (i, 0)),
        ],
        out_shape=[
            jax.ShapeDtypeStruct((rows2, LANE), f32),
            jax.ShapeDtypeStruct((rows2, LANE), f32),
        ],
        scratch_shapes=[pltpu.SMEM((8,), f32)],
    )(prelu_a, x, face_feats, W_e1, b_e1r, gam, bet, W_e2, b_e2r,
      W0, Wg, Wface, Wf)

    s2, lin2 = s_col, lin_col
    dinv2, t2 = pl.pallas_call(
        _post_body,
        out_shape=[
            jax.ShapeDtypeStruct((rows2, LANE), f32),
            jax.ShapeDtypeStruct((rows2, LANE), f32),
        ],
    )(hist.reshape(NC, rows2, LANE), s2)

    acc = _make_edge_sc(np_pad, rows_main, n_chunks, tail_rows)(
        edge_index, t2.reshape(np_pad), zeros_np)

    out2 = pl.pallas_call(
        _final_body,
        out_shape=jax.ShapeDtypeStruct((rows2, LANE), f32),
        in_specs=[
            pl.BlockSpec((rows2, LANE), lambda: (0, 0)),
            pl.BlockSpec((rows2, LANE), lambda: (0, 0)),
            pl.BlockSpec((rows2, LANE), lambda: (0, 0)),
            pl.BlockSpec((NC, rows2, LANE), lambda: (0, 0, 0)),
            pl.BlockSpec((1, 32), lambda: (0, 0)),
            pl.BlockSpec((1, 32), lambda: (0, 0)),
            pl.BlockSpec(memory_space=pltpu.SMEM),
        ],
    )(lin2, s2, dinv2, acc.reshape(NC, rows2, LANE), bgr, wft, bb)

    return out2.reshape(np_pad)[:n]


# speedup vs baseline: 1.4227x; 1.0243x over previous
import functools

import jax
import jax.numpy as jnp
from jax import lax
from jax.experimental import pallas as pl
from jax.experimental.pallas import tpu as pltpu
from jax.experimental.pallas import tpu_sc as plsc

NC = 2
NS = 16
NW = NC * NS
LANE = 128
CR = 130


def _main_body(n_rows, bn, pa_ref, x_ref, face_ref, we1_ref, be1_ref,
               gam_ref, bet_ref, we2_ref, be2_ref, w0_ref, wg_ref, wface_ref,
               wf_ref, s_out, lin_out, acc_ref):
    ph = pl.program_id(0)
    i = pl.program_id(1)

    @pl.when((ph == 0) & (i == 0))
    def _():
        for k in range(5):
            acc_ref[k] = 0.0

    @pl.when(ph == 0)
    def _():
        xb = x_ref[...]
        row = lax.broadcasted_iota(jnp.int32, xb.shape, 0) + i * bn
        valid = row < n_rows
        xa = jnp.concatenate(
            [jnp.where(valid, xb, 0.0),
             jnp.where(valid[:, 0:1], 1.0, 0.0)], axis=1)
        g = lax.dot_general(xa, xa, (((0,), (0,)), ((), ())),
                            preferred_element_type=jnp.float32)
        acc_ref[0] += g[0, 2]
        acc_ref[1] += g[1, 2]
        acc_ref[2] += g[0, 0]
        acc_ref[3] += g[0, 1]
        acc_ref[4] += g[1, 1]

    @pl.when(ph == 1)
    def _():
        ninv = 1.0 / float(n_rows)
        s0 = acc_ref[0] * ninv
        s1 = acc_ref[1] * ninv
        c00 = acc_ref[2] * ninv - s0 * s0
        c01 = acc_ref[3] * ninv - s0 * s1
        c11 = acc_ref[4] * ninv - s1 * s1
        w0r = we1_ref[0:1, :]
        w1r = we1_ref[1:2, :]
        mu = s0 * w0r + s1 * w1r + be1_ref[...]
        var = w0r * w0r * c00 + 2.0 * w0r * w1r * c01 + w1r * w1r * c11
        inv = lax.rsqrt(var + 1e-5)
        h = jnp.dot(x_ref[...], we1_ref[...],
                    preferred_element_type=jnp.float32) + be1_ref[...]
        hn = (h - mu) * (inv * gam_ref[...]) + bet_ref[...]
        a = pa_ref[0]
        hp = jnp.where(hn >= 0, hn, a * hn)
        e = jnp.dot(hp, we2_ref[...],
                    preferred_element_type=jnp.float32) + be2_ref[...]
        lin = jnp.dot(e, w0_ref[...], preferred_element_type=jnp.float32)
        lin_out[...] = lin.reshape(lin_out.shape)
        m = jnp.dot(e, wg_ref[...], preferred_element_type=jnp.float32)
        m = m + jnp.dot(face_ref[...], wface_ref[...],
                        preferred_element_type=jnp.float32)
        sv = jnp.dot(m, wf_ref[...], preferred_element_type=jnp.float32)
        s_out[...] = sv.reshape(s_out.shape)


def _post_body(hist_ref, s_ref, dinv_ref, t_ref):
    deg = hist_ref[0] + hist_ref[1] + 1.0
    dinv = lax.rsqrt(deg)
    dinv_ref[...] = dinv
    t_ref[...] = s_ref[...] * dinv


def _final_body(lin_ref, s_ref, dinv_ref, acc_ref, bg_ref, wft_ref, bb_ref,
                out_ref):
    const = jnp.sum(bg_ref[...] * wft_ref[...]) + bb_ref[0] + bb_ref[1]
    dinv = dinv_ref[...]
    out_ref[...] = (lin_ref[...] + (acc_ref[0] + acc_ref[1]) * dinv
                    + s_ref[...] * dinv * dinv + const)


def _make_hist_sc(np_pad, rows_main, n_chunks, tail_rows):
    mesh = plsc.VectorSubcoreMesh(core_axis_name="c", subcore_axis_name="s")

    @functools.partial(
        pl.kernel,
        out_type=jax.ShapeDtypeStruct((NC, np_pad), jnp.float32),
        mesh=mesh,
        scratch_types=[
            pltpu.VMEM((CR * LANE,), jnp.int32),
            pltpu.VMEM((CR * LANE,), jnp.float32),
            pltpu.VMEM((LANE,), jnp.int32),
            pltpu.VMEM((LANE,), jnp.float32),
            pltpu.VMEM_SHARED((np_pad,), jnp.float32),
        ],
    )
    def hist_sc(ei, ones_hbm, zeros_np, out, idx_v, ones_v, tidx_v,
                tones_v, hist_s):
        c = lax.axis_index("c")
        s = lax.axis_index("s")
        wid = c * NS + s

        @pl.when(s == 0)
        def _():
            pltpu.sync_copy(zeros_np, hist_s)

        pltpu.sync_copy(ones_hbm, ones_v)
        pltpu.sync_copy(ones_hbm.at[pl.ds(0, LANE)], tones_v)
        plsc.subcore_barrier()

        base = wid * rows_main * LANE

        def chunk(i, carry):
            off = base + i * (CR * LANE)
            pltpu.sync_copy(ei.at[1, pl.ds(off, CR * LANE)], idx_v)
            pltpu.sync_copy(ones_v, hist_s.at[idx_v], add=True)
            return carry

        lax.fori_loop(0, n_chunks, chunk, 0)

        @pl.when(wid < tail_rows)
        def _():
            toff = (NW * rows_main + wid) * LANE
            pltpu.sync_copy(ei.at[1, pl.ds(toff, LANE)], tidx_v)
            pltpu.sync_copy(tones_v, hist_s.at[tidx_v], add=True)

        plsc.subcore_barrier()

        @pl.when(s == 0)
        def _():
            pltpu.sync_copy(hist_s, out.at[c])

    return hist_sc


CRE = 65


def _make_edge_sc(np_pad, rows_main, tail_rows):
    mesh = plsc.VectorSubcoreMesh(core_axis_name="c", subcore_axis_name="s")
    n_chunks = rows_main // CRE
    crl = CRE * LANE

    @functools.partial(
        pl.kernel,
        out_type=jax.ShapeDtypeStruct((NC, np_pad), jnp.float32),
        mesh=mesh,
        scratch_types=[
            pltpu.VMEM((crl,), jnp.int32),
            pltpu.VMEM((crl,), jnp.int32),
            pltpu.VMEM((crl,), jnp.int32),
            pltpu.VMEM((crl,), jnp.int32),
            pltpu.VMEM((crl,), jnp.int32),
            pltpu.VMEM((crl,), jnp.int32),
            pltpu.VMEM((crl,), jnp.float32),
            pltpu.VMEM((crl,), jnp.float32),
            pltpu.VMEM((LANE,), jnp.int32),
            pltpu.VMEM((LANE,), jnp.int32),
            pltpu.VMEM((LANE,), jnp.float32),
            pltpu.VMEM_SHARED((np_pad,), jnp.float32),
            pltpu.VMEM_SHARED((np_pad,), jnp.float32),
            pltpu.SemaphoreType.DMA,
            pltpu.SemaphoreType.DMA,
        ],
    )
    def edge_sc(ei, t_hbm, zeros_np, out, s0, s1, s2, d0, d1, d2, v0, v1,
                tsidx_v, tdidx_v, tvals_v, t_s, acc_s, lsem, ssem):
        c = lax.axis_index("c")
        s = lax.axis_index("s")
        wid = c * NS + s

        @pl.when(s == 0)
        def _():
            pltpu.sync_copy(zeros_np, acc_s)

        @pl.when(s == 1)
        def _():
            pltpu.sync_copy(t_hbm, t_s)

        plsc.subcore_barrier()
        base = wid * rows_main * LANE
        sb = [s0, s1, s2]
        db = [d0, d1, d2]
        vb = [v0, v1]

        def start_loads(i):
            off = base + i * crl
            return (
                pltpu.async_copy(ei.at[0, pl.ds(off, crl)], sb[i % 3], lsem),
                pltpu.async_copy(ei.at[1, pl.ds(off, crl)], db[i % 3], lsem),
            )

        loads = {0: start_loads(0)}
        scat = {}
        for i in range(n_chunks):
            if i >= 2:
                scat[i - 2].wait()
            if i + 1 < n_chunks:
                loads[i + 1] = start_loads(i + 1)
            loads[i][0].wait()
            loads[i][1].wait()
            pltpu.sync_copy(t_s.at[sb[i % 3]], vb[i % 2])
            scat[i] = pltpu.async_copy(vb[i % 2], acc_s.at[db[i % 3]], ssem,
                                       add=True)
        for i in range(max(0, n_chunks - 2), n_chunks):
            scat[i].wait()

        @pl.when(wid < tail_rows)
        def _():
            toff = (NW * rows_main + wid) * LANE
            pltpu.sync_copy(ei.at[0, pl.ds(toff, LANE)], tsidx_v)
            pltpu.sync_copy(ei.at[1, pl.ds(toff, LANE)], tdidx_v)
            pltpu.sync_copy(t_s.at[tsidx_v], tvals_v)
            pltpu.sync_copy(tvals_v, acc_s.at[tdidx_v], add=True)

        plsc.subcore_barrier()

        @pl.when(s == 0)
        def _():
            pltpu.sync_copy(acc_s, out.at[c])

    return edge_sc


def kernel(x, edge_index, face_feats, W_e1, b_e1, bn_gamma, bn_beta, prelu_a,
           W_e2, b_e2, W0, b0, Wg, bg, Wface, Wf, bf):
    n = x.shape[0]
    e = edge_index.shape[1]
    ff = face_feats.shape[1]
    np_pad = ((n + 8 * LANE - 1) // (8 * LANE)) * (8 * LANE)
    rows2 = np_pad // LANE

    e_rows = e // LANE
    rows_main = (e_rows // NW // CR) * CR
    n_chunks = rows_main // CR
    tail_rows = e_rows - NW * rows_main
    assert rows_main * NW + tail_rows == e_rows and tail_rows <= NW

    bn = 7168
    brows = bn // LANE
    n_blocks = np_pad // bn
    hrows = bn * 32 // LANE
    hrows2 = np_pad * 32 // LANE

    f32 = jnp.float32
    b_e1r = b_e1.reshape(1, -1)
    gam = bn_gamma.reshape(1, -1)
    bet = bn_beta.reshape(1, -1)
    b_e2r = b_e2.reshape(1, -1)
    bgr = bg.reshape(1, -1)
    wft = Wf.reshape(1, -1)
    bb = jnp.concatenate([bf, b0]).astype(f32)
    zeros_np = jnp.zeros((np_pad,), f32)
    ones_cr = jnp.ones((CR * LANE,), f32)

    hist = _make_hist_sc(np_pad, rows_main, n_chunks, tail_rows)(
        edge_index, ones_cr, zeros_np)

    wspec = lambda shp: pl.BlockSpec(shp, lambda p, i: (0, 0))
    s_col, lin_col = pl.pallas_call(
        functools.partial(_main_body, n, bn),
        grid=(2, n_blocks),
        in_specs=[
            pl.BlockSpec(memory_space=pltpu.SMEM),
            pl.BlockSpec((bn, 2), lambda p, i: (i, 0)),
            pl.BlockSpec((bn, ff), lambda p, i: (i * p, 0)),
            wspec((2, 32)), wspec((1, 32)), wspec((1, 32)), wspec((1, 32)),
            wspec((32, 32)), wspec((1, 32)), wspec((32, 1)), wspec((32, 32)),
            wspec((ff, 32)), wspec((32, 1)),
        ],
        out_specs=[
            pl.BlockSpec((brows, LANE), lambda p, i: (i, 0)),
            pl.BlockSpec((brows, LANE), lambda p, i: (i, 0)),
        ],
        out_shape=[
            jax.ShapeDtypeStruct((rows2, LANE), f32),
            jax.ShapeDtypeStruct((rows2, LANE), f32),
        ],
        scratch_shapes=[pltpu.SMEM((8,), f32)],
    )(prelu_a, x, face_feats, W_e1, b_e1r, gam, bet, W_e2, b_e2r,
      W0, Wg, Wface, Wf)

    s2, lin2 = s_col, lin_col
    dinv2, t2 = pl.pallas_call(
        _post_body,
        out_shape=[
            jax.ShapeDtypeStruct((rows2, LANE), f32),
            jax.ShapeDtypeStruct((rows2, LANE), f32),
        ],
    )(hist.reshape(NC, rows2, LANE), s2)

    acc = _make_edge_sc(np_pad, rows_main, tail_rows)(
        edge_index, t2.reshape(np_pad), zeros_np)

    out2 = pl.pallas_call(
        _final_body,
        out_shape=jax.ShapeDtypeStruct((rows2, LANE), f32),
        in_specs=[
            pl.BlockSpec((rows2, LANE), lambda: (0, 0)),
            pl.BlockSpec((rows2, LANE), lambda: (0, 0)),
            pl.BlockSpec((rows2, LANE), lambda: (0, 0)),
            pl.BlockSpec((NC, rows2, LANE), lambda: (0, 0, 0)),
            pl.BlockSpec((1, 32), lambda: (0, 0)),
            pl.BlockSpec((1, 32), lambda: (0, 0)),
            pl.BlockSpec(memory_space=pltpu.SMEM),
        ],
    )(lin2, s2, dinv2, acc.reshape(NC, rows2, LANE), bgr, wft, bb)

    return out2.reshape(np_pad)[:n]
